# baseline (speedup 1.0000x reference)
import functools

import jax
import jax.numpy as jnp
from jax import lax
from jax.experimental import pallas as pl
from jax.experimental.pallas import tpu as pltpu
from jax.experimental.pallas import tpu_sc as plsc

KNB = 20
NGR = 8
_R = 128
_C = 1024
_RP = 128
_BIG = 2**30


def _knn_body(lo_ref, nt_ref, xr_ref, rlo_ref, rhi_ref, n2r_ref, xf_ref,
              n2t_ref, wa_ref, wu_ref, ba_ref, idx_ref, a_ref, u_ref):
    r = pl.program_id(0)
    xr = xr_ref[...]
    R = xr.shape[0]
    a_ref[...] = (jnp.dot(xr, wa_ref[...], preferred_element_type=jnp.float32)
                  + ba_ref[...])
    u_ref[...] = jnp.dot(xr, wu_ref[...], preferred_element_type=jnp.float32)

    n2r = n2r_ref[...]
    rlo = rlo_ref[...]
    rhi = rhi_ref[...]
    lo = lo_ref[r]
    nt = nt_ref[r]
    INF = jnp.float32(jnp.inf)
    sv0 = jnp.full((R, KNB), INF, jnp.float32)
    si0 = lax.broadcasted_iota(jnp.int32, (R, KNB), 1).astype(jnp.float32)
    W2 = (_C + KNB + KNB) // 2

    def tile(t, S):
        sv, si = S
        j0 = pl.multiple_of(lo + t * _C, 128)
        xc = xf_ref[pl.ds(j0, _C), :]
        n2c = n2t_ref[:, pl.ds(j0, _C)]
        mm = lax.dot_general(xr, xc, (((1,), (1,)), ((), ())),
                             preferred_element_type=jnp.float32)
        d = (n2r + n2c) - 2.0 * mm
        ci = lax.broadcasted_iota(jnp.int32, (R, _C), 1) + j0
        ok = (ci >= rlo) & (ci < rhi)
        d = jnp.where(ok, d, INF)
        cif = ci.astype(jnp.float32)
        V = jnp.concatenate([d, sv, jnp.full((R, KNB), INF, jnp.float32)],
                            axis=1)
        I = jnp.concatenate([cif, si, jnp.full((R, KNB), 3.0e7, jnp.float32)],
                            axis=1)
        A, B = V[:, :W2], V[:, W2:]
        IA, IB = I[:, :W2], I[:, W2:]
        P = A <= B
        M = jnp.where(P, A, B)
        IM = jnp.where(P, IA, IB)
        L = jnp.where(P, B, A)
        IL = jnp.where(P, IB, IA)
        vs, ixs = [], []
        for _ in range(KNB):
            v = jnp.min(M, axis=1, keepdims=True)
            eq = M == v
            ix = jnp.min(jnp.where(eq, IM, 3.0e7), axis=1, keepdims=True)
            kill = IM == ix
            M = jnp.where(kill, L, M)
            IM = jnp.where(kill, IL, IM)
            L = jnp.where(kill, INF, L)
            vs.append(v)
            ixs.append(ix)
        return jnp.concatenate(vs, axis=1), jnp.concatenate(ixs, axis=1)

    _, si = lax.fori_loop(0, nt, tile, (sv0, si0))
    idx_ref[...] = si.astype(jnp.int32)


def _knn_call(x, rlo, rhi, lo128, nt, wa, wu, ba):
    N, D = x.shape
    Fa = wa.shape[1]
    Fu = wu.shape[1]
    G = N // _R
    NP = N + _C
    xpad = jnp.concatenate([x, jnp.zeros((_C, D), x.dtype)], axis=0)
    n2 = jnp.sum(x * x, axis=1)
    n2r = n2[:, None]
    n2t = jnp.concatenate([n2, jnp.zeros((_C,), n2.dtype)])[None, :]
    grid_spec = pltpu.PrefetchScalarGridSpec(
        num_scalar_prefetch=2,
        grid=(G,),
        in_specs=[
            pl.BlockSpec((_R, D), lambda r, lo_, nt_: (r, 0)),
            pl.BlockSpec((_R, 1), lambda r, lo_, nt_: (r, 0)),
            pl.BlockSpec((_R, 1), lambda r, lo_, nt_: (r, 0)),
            pl.BlockSpec((_R, 1), lambda r, lo_, nt_: (r, 0)),
            pl.BlockSpec((NP, D), lambda r, lo_, nt_: (0, 0)),
            pl.BlockSpec((1, NP), lambda r, lo_, nt_: (0, 0)),
            pl.BlockSpec((D, Fa), lambda r, lo_, nt_: (0, 0)),
            pl.BlockSpec((D, Fu), lambda r, lo_, nt_: (0, 0)),
            pl.BlockSpec((1, Fa), lambda r, lo_, nt_: (0, 0)),
        ],
        out_specs=[
            pl.BlockSpec((_R, KNB), lambda r, lo_, nt_: (r, 0)),
            pl.BlockSpec((_R, Fa), lambda r, lo_, nt_: (r, 0)),
            pl.BlockSpec((_R, Fu), lambda r, lo_, nt_: (r, 0)),
        ],
    )
    return pl.pallas_call(
        _knn_body,
        grid_spec=grid_spec,
        out_shape=[
            jax.ShapeDtypeStruct((N, KNB), jnp.int32),
            jax.ShapeDtypeStruct((N, Fa), jnp.float32),
            jax.ShapeDtypeStruct((N, Fu), jnp.float32),
        ],
    )(lo128, nt, x, rlo, rhi, n2r, xpad, n2t, wa, wu, ba)


def _sc_gather(tab, idx_flat):
    (E,) = idx_flat.shape
    _, D = tab.shape
    NW = 32
    epw = E // NW
    CH = 128
    nch = epw // CH
    mesh = plsc.VectorSubcoreMesh(core_axis_name="c", subcore_axis_name="s")

    NBUF = 4
    ngrp = nch // NBUF

    @functools.partial(
        pl.kernel,
        out_type=jax.ShapeDtypeStruct((E, D), jnp.float32),
        mesh=mesh,
        scratch_types=[
            pltpu.VMEM((epw,), jnp.int32),
            pltpu.VMEM((NBUF, CH, D), jnp.float32),
            pltpu.SemaphoreType.DMA,
        ],
    )
    def gk(idx_hbm, tab_hbm, out_hbm, idx_v, rows_v, sem):
        wid = lax.axis_index("s") * 2 + lax.axis_index("c")
        base = wid * epw
        pltpu.sync_copy(idx_hbm.at[pl.ds(base, epw)], idx_v)

        def body(g, carry):
            ch0 = g * NBUF
            copies = [
                pltpu.async_copy(
                    tab_hbm.at[idx_v.at[pl.ds((ch0 + b) * CH, CH)]],
                    rows_v.at[b], sem)
                for b in range(NBUF)
            ]
            for b in range(NBUF):
                copies[b].wait()
            for b in range(NBUF):
                pltpu.sync_copy(rows_v.at[b],
                                out_hbm.at[pl.ds(base + (ch0 + b) * CH, CH)])
            return carry

        lax.fori_loop(0, ngrp, body, 0)

    return gk(idx_flat, tab)


def _edge1_body(e_ref, xi_ref, w1_ref, b1_ref, w2_ref, b2_ref, x1_ref):
    xi = xi_ref[...]
    RP = xi.shape[0]
    feats = []
    for k in range(KNB):
        xj = e_ref[k][:, :3]
        feats.append(jnp.concatenate([xi, xj - xi], axis=1))
    F = jnp.concatenate(feats, axis=0)
    H1 = jnp.maximum(
        jnp.dot(F, w1_ref[...], preferred_element_type=jnp.float32)
        + b1_ref[...], 0.0)
    H = jnp.dot(H1, w2_ref[...], preferred_element_type=jnp.float32)
    m = H[0:RP]
    for k in range(1, KNB):
        m = jnp.maximum(m, H[k * RP:(k + 1) * RP])
    x1_ref[...] = m + b2_ref[...]


def _final_body(g2_ref, c_ref, x1_ref, br_ref, wla_ref, wlb_ref, bl_ref, out_ref):
    p = pl.program_id(0)
    m = g2_ref[0]
    for k in range(1, KNB):
        m = jnp.maximum(m, g2_ref[k])
    x2 = c_ref[...] + m
    h = (jnp.dot(x1_ref[...], wla_ref[...], preferred_element_type=jnp.float32)
         + jnp.dot(x2, wlb_ref[...], preferred_element_type=jnp.float32)
         + bl_ref[...])
    NINF = jnp.float32(-jnp.inf)

    @pl.when(p == 0)
    def _():
        out_ref[...] = jnp.full(out_ref.shape, NINF, jnp.float32)

    b = br_ref[...]
    rows = [jnp.max(jnp.where(b == g, h, NINF), axis=0, keepdims=True)
            for g in range(NGR)]
    out_ref[...] = jnp.maximum(out_ref[...], jnp.concatenate(rows, axis=0))


def kernel(pos, batch, W1, b1, W2, b2, W3, b3, Wl, bl, train=0):
    N = pos.shape[0]
    batch = batch.astype(jnp.int32)
    starts = jnp.searchsorted(
        batch, jnp.arange(NGR + 1, dtype=jnp.int32), side="left").astype(jnp.int32)
    rlo = starts[batch][:, None]
    rhi = starts[batch + 1][:, None]
    lo_blk = starts[batch[::_R]]
    hi_blk = starts[batch[_R - 1::_R] + 1]
    lo128 = (lo_blk // 128) * 128
    nt = (hi_blk - lo128 + _C - 1) // _C

    W1a = W1[:3] - W1[3:]
    W1u = W1[3:]
    idx1, _, _ = _knn_call(pos, rlo, rhi, lo128, nt, W1a, W1u, b1[None, :])
    pos_pad = jnp.concatenate([pos, jnp.zeros((N, 125), pos.dtype)], axis=1)
    e1 = _sc_gather(pos_pad, idx1.T.reshape(-1))
    x1 = pl.pallas_call(
        _edge1_body,
        grid=(N // _RP,),
        in_specs=[
            pl.BlockSpec((KNB, _RP, 128), lambda p: (0, p, 0)),
            pl.BlockSpec((_RP, 3), lambda p: (p, 0)),
            pl.BlockSpec((6, 64), lambda p: (0, 0)),
            pl.BlockSpec((1, 64), lambda p: (0, 0)),
            pl.BlockSpec((64, 64), lambda p: (0, 0)),
            pl.BlockSpec((1, 64), lambda p: (0, 0)),
        ],
        out_specs=pl.BlockSpec((_RP, 64), lambda p: (p, 0)),
        out_shape=jax.ShapeDtypeStruct((N, 64), jnp.float32),
    )(e1.reshape(KNB, N, 128), pos, W1, b1[None, :], W2, b2[None, :])

    W3a = W3[:64] - W3[64:]
    W3u = W3[64:]
    idx2, c2, y2 = _knn_call(x1, rlo, rhi, lo128, nt, W3a, W3u, b3[None, :])
    g2 = _sc_gather(y2, idx2.T.reshape(-1))

    out = pl.pallas_call(
        _final_body,
        grid=(N // _RP,),
        in_specs=[
            pl.BlockSpec((KNB, _RP, 128), lambda p: (0, p, 0)),
            pl.BlockSpec((_RP, 128), lambda p: (p, 0)),
            pl.BlockSpec((_RP, 64), lambda p: (p, 0)),
            pl.BlockSpec((_RP, 1), lambda p: (p, 0)),
            pl.BlockSpec((64, 128), lambda p: (0, 0)),
            pl.BlockSpec((128, 128), lambda p: (0, 0)),
            pl.BlockSpec((1, 128), lambda p: (0, 0)),
        ],
        out_specs=pl.BlockSpec((NGR, 128), lambda p: (0, 0)),
        out_shape=jax.ShapeDtypeStruct((NGR, 128), jnp.float32),
    )(g2.reshape(KNB, N, 128), c2, x1, batch[:, None], Wl[:64], Wl[64:],
      bl[None, :])
    return out


# speedup vs baseline: 1.0391x; 1.0391x over previous
import functools

import jax
import jax.numpy as jnp
from jax import lax
from jax.experimental import pallas as pl
from jax.experimental.pallas import tpu as pltpu
from jax.experimental.pallas import tpu_sc as plsc

KNB = 20
NGR = 8
_R = 512
_C = 1024
_RP = 128
_BIG = 2**30


def _knn_body(lo_ref, nt_ref, xr_ref, rlo_ref, rhi_ref, n2r_ref, xf_ref,
              n2t_ref, wa_ref, wu_ref, ba_ref, idx_ref, a_ref, u_ref):
    r = pl.program_id(0)
    xr = xr_ref[...]
    R = xr.shape[0]
    a_ref[...] = (jnp.dot(xr, wa_ref[...], preferred_element_type=jnp.float32)
                  + ba_ref[...])
    u_ref[...] = jnp.dot(xr, wu_ref[...], preferred_element_type=jnp.float32)

    n2r = n2r_ref[...]
    rlo = rlo_ref[...]
    rhi = rhi_ref[...]
    lo = lo_ref[r]
    nt = nt_ref[r]
    INF = jnp.float32(jnp.inf)
    sv0 = jnp.full((R, KNB), INF, jnp.float32)
    si0 = lax.broadcasted_iota(jnp.int32, (R, KNB), 1).astype(jnp.float32)
    W2 = (_C + KNB + KNB) // 2

    def tile(t, S):
        sv, si = S
        j0 = pl.multiple_of(lo + t * _C, 128)
        xc = xf_ref[pl.ds(j0, _C), :]
        n2c = n2t_ref[:, pl.ds(j0, _C)]
        mm = lax.dot_general(xr, xc, (((1,), (1,)), ((), ())),
                             preferred_element_type=jnp.float32)
        d = (n2r + n2c) - 2.0 * mm
        ci = lax.broadcasted_iota(jnp.int32, (R, _C), 1) + j0
        ok = (ci >= rlo) & (ci < rhi)
        d = jnp.where(ok, d, INF)
        cif = ci.astype(jnp.float32)
        V = jnp.concatenate([d, sv, jnp.full((R, KNB), INF, jnp.float32)],
                            axis=1)
        I = jnp.concatenate([cif, si, jnp.full((R, KNB), 3.0e7, jnp.float32)],
                            axis=1)
        A, B = V[:, :W2], V[:, W2:]
        IA, IB = I[:, :W2], I[:, W2:]
        P = A <= B
        M = jnp.where(P, A, B)
        IM = jnp.where(P, IA, IB)
        L = jnp.where(P, B, A)
        IL = jnp.where(P, IB, IA)
        vs, ixs = [], []
        for _ in range(KNB):
            v = jnp.min(M, axis=1, keepdims=True)
            eq = M == v
            ix = jnp.min(jnp.where(eq, IM, 3.0e7), axis=1, keepdims=True)
            kill = IM == ix
            M = jnp.where(kill, L, M)
            IM = jnp.where(kill, IL, IM)
            L = jnp.where(kill, INF, L)
            vs.append(v)
            ixs.append(ix)
        return jnp.concatenate(vs, axis=1), jnp.concatenate(ixs, axis=1)

    _, si = lax.fori_loop(0, nt, tile, (sv0, si0))
    idx_ref[...] = si.astype(jnp.int32)


def _knn_call(x, rlo, rhi, lo128, nt, wa, wu, ba):
    N, D = x.shape
    Fa = wa.shape[1]
    Fu = wu.shape[1]
    G = N // _R
    NP = N + _C
    xpad = jnp.concatenate([x, jnp.zeros((_C, D), x.dtype)], axis=0)
    n2 = jnp.sum(x * x, axis=1)
    n2r = n2[:, None]
    n2t = jnp.concatenate([n2, jnp.zeros((_C,), n2.dtype)])[None, :]
    grid_spec = pltpu.PrefetchScalarGridSpec(
        num_scalar_prefetch=2,
        grid=(G,),
        in_specs=[
            pl.BlockSpec((_R, D), lambda r, lo_, nt_: (r, 0)),
            pl.BlockSpec((_R, 1), lambda r, lo_, nt_: (r, 0)),
            pl.BlockSpec((_R, 1), lambda r, lo_, nt_: (r, 0)),
            pl.BlockSpec((_R, 1), lambda r, lo_, nt_: (r, 0)),
            pl.BlockSpec((NP, D), lambda r, lo_, nt_: (0, 0)),
            pl.BlockSpec((1, NP), lambda r, lo_, nt_: (0, 0)),
            pl.BlockSpec((D, Fa), lambda r, lo_, nt_: (0, 0)),
            pl.BlockSpec((D, Fu), lambda r, lo_, nt_: (0, 0)),
            pl.BlockSpec((1, Fa), lambda r, lo_, nt_: (0, 0)),
        ],
        out_specs=[
            pl.BlockSpec((_R, KNB), lambda r, lo_, nt_: (r, 0)),
            pl.BlockSpec((_R, Fa), lambda r, lo_, nt_: (r, 0)),
            pl.BlockSpec((_R, Fu), lambda r, lo_, nt_: (r, 0)),
        ],
    )
    return pl.pallas_call(
        _knn_body,
        grid_spec=grid_spec,
        out_shape=[
            jax.ShapeDtypeStruct((N, KNB), jnp.int32),
            jax.ShapeDtypeStruct((N, Fa), jnp.float32),
            jax.ShapeDtypeStruct((N, Fu), jnp.float32),
        ],
    )(lo128, nt, x, rlo, rhi, n2r, xpad, n2t, wa, wu, ba)


def _sc_gather(tab, idx_flat):
    (E,) = idx_flat.shape
    _, D = tab.shape
    NW = 32
    epw = E // NW
    CH = 128
    nch = epw // CH
    mesh = plsc.VectorSubcoreMesh(core_axis_name="c", subcore_axis_name="s")

    NBUF = 4
    ngrp = nch // NBUF

    @functools.partial(
        pl.kernel,
        out_type=jax.ShapeDtypeStruct((E, D), jnp.float32),
        mesh=mesh,
        scratch_types=[
            pltpu.VMEM((epw,), jnp.int32),
            pltpu.VMEM((NBUF, CH, D), jnp.float32),
            pltpu.SemaphoreType.DMA,
        ],
    )
    def gk(idx_hbm, tab_hbm, out_hbm, idx_v, rows_v, sem):
        wid = lax.axis_index("s") * 2 + lax.axis_index("c")
        base = wid * epw
        pltpu.sync_copy(idx_hbm.at[pl.ds(base, epw)], idx_v)

        def body(g, carry):
            ch0 = g * NBUF
            copies = [
                pltpu.async_copy(
                    tab_hbm.at[idx_v.at[pl.ds((ch0 + b) * CH, CH)]],
                    rows_v.at[b], sem)
                for b in range(NBUF)
            ]
            for b in range(NBUF):
                copies[b].wait()
            for b in range(NBUF):
                pltpu.sync_copy(rows_v.at[b],
                                out_hbm.at[pl.ds(base + (ch0 + b) * CH, CH)])
            return carry

        lax.fori_loop(0, ngrp, body, 0)

    return gk(idx_flat, tab)


def _edge1_body(e_ref, xi_ref, w1_ref, b1_ref, w2_ref, b2_ref, x1_ref):
    xi = xi_ref[...]
    RP = xi.shape[0]
    feats = []
    for k in range(KNB):
        xj = e_ref[k][:, :3]
        feats.append(jnp.concatenate([xi, xj - xi], axis=1))
    F = jnp.concatenate(feats, axis=0)
    H1 = jnp.maximum(
        jnp.dot(F, w1_ref[...], preferred_element_type=jnp.float32)
        + b1_ref[...], 0.0)
    H = jnp.dot(H1, w2_ref[...], preferred_element_type=jnp.float32)
    m = H[0:RP]
    for k in range(1, KNB):
        m = jnp.maximum(m, H[k * RP:(k + 1) * RP])
    x1_ref[...] = m + b2_ref[...]


def _final_body(g2_ref, c_ref, x1_ref, br_ref, wla_ref, wlb_ref, bl_ref, out_ref):
    p = pl.program_id(0)
    m = g2_ref[0]
    for k in range(1, KNB):
        m = jnp.maximum(m, g2_ref[k])
    x2 = c_ref[...] + m
    h = (jnp.dot(x1_ref[...], wla_ref[...], preferred_element_type=jnp.float32)
         + jnp.dot(x2, wlb_ref[...], preferred_element_type=jnp.float32)
         + bl_ref[...])
    NINF = jnp.float32(-jnp.inf)

    @pl.when(p == 0)
    def _():
        out_ref[...] = jnp.full(out_ref.shape, NINF, jnp.float32)

    b = br_ref[...]
    rows = [jnp.max(jnp.where(b == g, h, NINF), axis=0, keepdims=True)
            for g in range(NGR)]
    out_ref[...] = jnp.maximum(out_ref[...], jnp.concatenate(rows, axis=0))


def kernel(pos, batch, W1, b1, W2, b2, W3, b3, Wl, bl, train=0):
    N = pos.shape[0]
    batch = batch.astype(jnp.int32)
    starts = jnp.searchsorted(
        batch, jnp.arange(NGR + 1, dtype=jnp.int32), side="left").astype(jnp.int32)
    rlo = starts[batch][:, None]
    rhi = starts[batch + 1][:, None]
    lo_blk = starts[batch[::_R]]
    hi_blk = starts[batch[_R - 1::_R] + 1]
    lo128 = (lo_blk // 128) * 128
    nt = (hi_blk - lo128 + _C - 1) // _C

    W1a = W1[:3] - W1[3:]
    W1u = W1[3:]
    idx1, _, _ = _knn_call(pos, rlo, rhi, lo128, nt, W1a, W1u, b1[None, :])
    pos_pad = jnp.concatenate([pos, jnp.zeros((N, 125), pos.dtype)], axis=1)
    e1 = _sc_gather(pos_pad, idx1.T.reshape(-1))
    x1 = pl.pallas_call(
        _edge1_body,
        grid=(N // _RP,),
        in_specs=[
            pl.BlockSpec((KNB, _RP, 128), lambda p: (0, p, 0)),
            pl.BlockSpec((_RP, 3), lambda p: (p, 0)),
            pl.BlockSpec((6, 64), lambda p: (0, 0)),
            pl.BlockSpec((1, 64), lambda p: (0, 0)),
            pl.BlockSpec((64, 64), lambda p: (0, 0)),
            pl.BlockSpec((1, 64), lambda p: (0, 0)),
        ],
        out_specs=pl.BlockSpec((_RP, 64), lambda p: (p, 0)),
        out_shape=jax.ShapeDtypeStruct((N, 64), jnp.float32),
    )(e1.reshape(KNB, N, 128), pos, W1, b1[None, :], W2, b2[None, :])

    W3a = W3[:64] - W3[64:]
    W3u = W3[64:]
    idx2, c2, y2 = _knn_call(x1, rlo, rhi, lo128, nt, W3a, W3u, b3[None, :])
    g2 = _sc_gather(y2, idx2.T.reshape(-1))

    out = pl.pallas_call(
        _final_body,
        grid=(N // _RP,),
        in_specs=[
            pl.BlockSpec((KNB, _RP, 128), lambda p: (0, p, 0)),
            pl.BlockSpec((_RP, 128), lambda p: (p, 0)),
            pl.BlockSpec((_RP, 64), lambda p: (p, 0)),
            pl.BlockSpec((_RP, 1), lambda p: (p, 0)),
            pl.BlockSpec((64, 128), lambda p: (0, 0)),
            pl.BlockSpec((128, 128), lambda p: (0, 0)),
            pl.BlockSpec((1, 128), lambda p: (0, 0)),
        ],
        out_specs=pl.BlockSpec((NGR, 128), lambda p: (0, 0)),
        out_shape=jax.ShapeDtypeStruct((NGR, 128), jnp.float32),
    )(g2.reshape(KNB, N, 128), c2, x1, batch[:, None], Wl[:64], Wl[64:],
      bl[None, :])
    return out


# speedup vs baseline: 1.1668x; 1.1229x over previous
import functools

import jax
import jax.numpy as jnp
from jax import lax
from jax.experimental import pallas as pl
from jax.experimental.pallas import tpu as pltpu
from jax.experimental.pallas import tpu_sc as plsc

KNB = 20
NGR = 8
_R = 256
_C = 768
_RP = 128
_BIG = 2**30


def _knn_body(lo_ref, nt_ref, xr_ref, rlo_ref, rhi_ref, n2r_ref, xf_ref,
              n2t_ref, wa_ref, wu_ref, ba_ref, idx_ref, a_ref, u_ref):
    r = pl.program_id(0)
    xr = xr_ref[...]
    R = xr.shape[0]
    a_ref[...] = (jnp.dot(xr, wa_ref[...], preferred_element_type=jnp.float32)
                  + ba_ref[...])
    u_ref[...] = jnp.dot(xr, wu_ref[...], preferred_element_type=jnp.float32)

    n2r = n2r_ref[...]
    rlo = rlo_ref[...]
    rhi = rhi_ref[...]
    lo = lo_ref[r]
    nt = nt_ref[r]
    INF = jnp.float32(jnp.inf)
    sv0 = jnp.full((R, KNB), INF, jnp.float32)
    si0 = lax.broadcasted_iota(jnp.int32, (R, KNB), 1).astype(jnp.float32)
    W2 = (_C + KNB + KNB) // 2

    def tile(t, S):
        sv, si = S
        j0 = pl.multiple_of(lo + t * _C, 128)
        xc = xf_ref[pl.ds(j0, _C), :]
        n2c = n2t_ref[:, pl.ds(j0, _C)]
        mm = lax.dot_general(xr, xc, (((1,), (1,)), ((), ())),
                             preferred_element_type=jnp.float32)
        d = (n2r + n2c) - 2.0 * mm
        ci = lax.broadcasted_iota(jnp.int32, (R, _C), 1) + j0
        ok = (ci >= rlo) & (ci < rhi)
        d = jnp.where(ok, d, INF)
        cif = ci.astype(jnp.float32)
        V = jnp.concatenate([d, sv, jnp.full((R, KNB), INF, jnp.float32)],
                            axis=1)
        I = jnp.concatenate([cif, si, jnp.full((R, KNB), 3.0e7, jnp.float32)],
                            axis=1)
        A, B = V[:, :W2], V[:, W2:]
        IA, IB = I[:, :W2], I[:, W2:]
        P = A <= B
        M = jnp.where(P, A, B)
        IM = jnp.where(P, IA, IB)
        L = jnp.where(P, B, A)
        IL = jnp.where(P, IB, IA)
        vs, ixs = [], []
        for _ in range(KNB):
            v = jnp.min(M, axis=1, keepdims=True)
            eq = M == v
            ix = jnp.min(jnp.where(eq, IM, 3.0e7), axis=1, keepdims=True)
            kill = IM == ix
            M = jnp.where(kill, L, M)
            IM = jnp.where(kill, IL, IM)
            L = jnp.where(kill, INF, L)
            vs.append(v)
            ixs.append(ix)
        return jnp.concatenate(vs, axis=1), jnp.concatenate(ixs, axis=1)

    _, si = lax.fori_loop(0, nt, tile, (sv0, si0))
    idx_ref[...] = si.astype(jnp.int32)


def _knn_call(x, rlo, rhi, lo128, nt, wa, wu, ba):
    N, D = x.shape
    Fa = wa.shape[1]
    Fu = wu.shape[1]
    G = N // _R
    NP = N + _C
    xpad = jnp.concatenate([x, jnp.zeros((_C, D), x.dtype)], axis=0)
    n2 = jnp.sum(x * x, axis=1)
    n2r = n2[:, None]
    n2t = jnp.concatenate([n2, jnp.zeros((_C,), n2.dtype)])[None, :]
    grid_spec = pltpu.PrefetchScalarGridSpec(
        num_scalar_prefetch=2,
        grid=(G,),
        in_specs=[
            pl.BlockSpec((_R, D), lambda r, lo_, nt_: (r, 0)),
            pl.BlockSpec((_R, 1), lambda r, lo_, nt_: (r, 0)),
            pl.BlockSpec((_R, 1), lambda r, lo_, nt_: (r, 0)),
            pl.BlockSpec((_R, 1), lambda r, lo_, nt_: (r, 0)),
            pl.BlockSpec((NP, D), lambda r, lo_, nt_: (0, 0)),
            pl.BlockSpec((1, NP), lambda r, lo_, nt_: (0, 0)),
            pl.BlockSpec((D, Fa), lambda r, lo_, nt_: (0, 0)),
            pl.BlockSpec((D, Fu), lambda r, lo_, nt_: (0, 0)),
            pl.BlockSpec((1, Fa), lambda r, lo_, nt_: (0, 0)),
        ],
        out_specs=[
            pl.BlockSpec((_R, KNB), lambda r, lo_, nt_: (r, 0)),
            pl.BlockSpec((_R, Fa), lambda r, lo_, nt_: (r, 0)),
            pl.BlockSpec((_R, Fu), lambda r, lo_, nt_: (r, 0)),
        ],
    )
    return pl.pallas_call(
        _knn_body,
        grid_spec=grid_spec,
        out_shape=[
            jax.ShapeDtypeStruct((N, KNB), jnp.int32),
            jax.ShapeDtypeStruct((N, Fa), jnp.float32),
            jax.ShapeDtypeStruct((N, Fu), jnp.float32),
        ],
    )(lo128, nt, x, rlo, rhi, n2r, xpad, n2t, wa, wu, ba)


def _sc_gather(tab, idx_flat):
    (E,) = idx_flat.shape
    _, D = tab.shape
    NW = 32
    epw = E // NW
    CH = 128
    nch = epw // CH
    mesh = plsc.VectorSubcoreMesh(core_axis_name="c", subcore_axis_name="s")

    NBUF = 4
    ngrp = nch // NBUF

    @functools.partial(
        pl.kernel,
        out_type=jax.ShapeDtypeStruct((E, D), jnp.float32),
        mesh=mesh,
        scratch_types=[
            pltpu.VMEM((epw,), jnp.int32),
            pltpu.VMEM((NBUF, CH, D), jnp.float32),
            pltpu.SemaphoreType.DMA,
        ],
    )
    def gk(idx_hbm, tab_hbm, out_hbm, idx_v, rows_v, sem):
        wid = lax.axis_index("s") * 2 + lax.axis_index("c")
        base = wid * epw
        pltpu.sync_copy(idx_hbm.at[pl.ds(base, epw)], idx_v)

        def body(g, carry):
            ch0 = g * NBUF
            copies = [
                pltpu.async_copy(
                    tab_hbm.at[idx_v.at[pl.ds((ch0 + b) * CH, CH)]],
                    rows_v.at[b], sem)
                for b in range(NBUF)
            ]
            for b in range(NBUF):
                copies[b].wait()
            for b in range(NBUF):
                pltpu.sync_copy(rows_v.at[b],
                                out_hbm.at[pl.ds(base + (ch0 + b) * CH, CH)])
            return carry

        lax.fori_loop(0, ngrp, body, 0)

    return gk(idx_flat, tab)


def _edge1_body(e_ref, xi_ref, w1_ref, b1_ref, w2_ref, b2_ref, x1_ref):
    xi = xi_ref[...]
    RP = xi.shape[0]
    feats = []
    for k in range(KNB):
        xj = e_ref[k][:, :3]
        feats.append(jnp.concatenate([xi, xj - xi], axis=1))
    F = jnp.concatenate(feats, axis=0)
    H1 = jnp.maximum(
        jnp.dot(F, w1_ref[...], preferred_element_type=jnp.float32)
        + b1_ref[...], 0.0)
    H = jnp.dot(H1, w2_ref[...], preferred_element_type=jnp.float32)
    m = H[0:RP]
    for k in range(1, KNB):
        m = jnp.maximum(m, H[k * RP:(k + 1) * RP])
    x1_ref[...] = m + b2_ref[...]


def _final_body(g2_ref, c_ref, x1_ref, br_ref, wla_ref, wlb_ref, bl_ref, out_ref):
    p = pl.program_id(0)
    m = g2_ref[0]
    for k in range(1, KNB):
        m = jnp.maximum(m, g2_ref[k])
    x2 = c_ref[...] + m
    h = (jnp.dot(x1_ref[...], wla_ref[...], preferred_element_type=jnp.float32)
         + jnp.dot(x2, wlb_ref[...], preferred_element_type=jnp.float32)
         + bl_ref[...])
    NINF = jnp.float32(-jnp.inf)

    @pl.when(p == 0)
    def _():
        out_ref[...] = jnp.full(out_ref.shape, NINF, jnp.float32)

    b = br_ref[...]
    rows = [jnp.max(jnp.where(b == g, h, NINF), axis=0, keepdims=True)
            for g in range(NGR)]
    out_ref[...] = jnp.maximum(out_ref[...], jnp.concatenate(rows, axis=0))


def kernel(pos, batch, W1, b1, W2, b2, W3, b3, Wl, bl, train=0):
    N = pos.shape[0]
    batch = batch.astype(jnp.int32)
    starts = jnp.searchsorted(
        batch, jnp.arange(NGR + 1, dtype=jnp.int32), side="left").astype(jnp.int32)
    rlo = starts[batch][:, None]
    rhi = starts[batch + 1][:, None]
    lo_blk = starts[batch[::_R]]
    hi_blk = starts[batch[_R - 1::_R] + 1]
    lo128 = (lo_blk // 128) * 128
    nt = (hi_blk - lo128 + _C - 1) // _C

    W1a = W1[:3] - W1[3:]
    W1u = W1[3:]
    idx1, _, _ = _knn_call(pos, rlo, rhi, lo128, nt, W1a, W1u, b1[None, :])
    pos_pad = jnp.concatenate([pos, jnp.zeros((N, 125), pos.dtype)], axis=1)
    e1 = _sc_gather(pos_pad, idx1.T.reshape(-1))
    x1 = pl.pallas_call(
        _edge1_body,
        grid=(N // _RP,),
        in_specs=[
            pl.BlockSpec((KNB, _RP, 128), lambda p: (0, p, 0)),
            pl.BlockSpec((_RP, 3), lambda p: (p, 0)),
            pl.BlockSpec((6, 64), lambda p: (0, 0)),
            pl.BlockSpec((1, 64), lambda p: (0, 0)),
            pl.BlockSpec((64, 64), lambda p: (0, 0)),
            pl.BlockSpec((1, 64), lambda p: (0, 0)),
        ],
        out_specs=pl.BlockSpec((_RP, 64), lambda p: (p, 0)),
        out_shape=jax.ShapeDtypeStruct((N, 64), jnp.float32),
    )(e1.reshape(KNB, N, 128), pos, W1, b1[None, :], W2, b2[None, :])

    W3a = W3[:64] - W3[64:]
    W3u = W3[64:]
    idx2, c2, y2 = _knn_call(x1, rlo, rhi, lo128, nt, W3a, W3u, b3[None, :])
    g2 = _sc_gather(y2, idx2.T.reshape(-1))

    out = pl.pallas_call(
        _final_body,
        grid=(N // _RP,),
        in_specs=[
            pl.BlockSpec((KNB, _RP, 128), lambda p: (0, p, 0)),
            pl.BlockSpec((_RP, 128), lambda p: (p, 0)),
            pl.BlockSpec((_RP, 64), lambda p: (p, 0)),
            pl.BlockSpec((_RP, 1), lambda p: (p, 0)),
            pl.BlockSpec((64, 128), lambda p: (0, 0)),
            pl.BlockSpec((128, 128), lambda p: (0, 0)),
            pl.BlockSpec((1, 128), lambda p: (0, 0)),
        ],
        out_specs=pl.BlockSpec((NGR, 128), lambda p: (0, 0)),
        out_shape=jax.ShapeDtypeStruct((NGR, 128), jnp.float32),
    )(g2.reshape(KNB, N, 128), c2, x1, batch[:, None], Wl[:64], Wl[64:],
      bl[None, :])
    return out


# speedup vs baseline: 1.2287x; 1.0531x over previous
import functools

import jax
import jax.numpy as jnp
from jax import lax
from jax.experimental import pallas as pl
from jax.experimental.pallas import tpu as pltpu
from jax.experimental.pallas import tpu_sc as plsc

KNB = 20
NGR = 8
_R = 256
_C = 768
_RP = 128
_BIG = 2**30


def _knn_body(lo_ref, nt_ref, xr_ref, rlo_ref, rhi_ref, n2r_ref, xf_ref,
              n2t_ref, idx_ref):
    r = pl.program_id(0)
    xr = xr_ref[...]
    R = xr.shape[0]
    n2r = n2r_ref[...]
    rlo = rlo_ref[...]
    rhi = rhi_ref[...]
    lo = lo_ref[r]
    nt = nt_ref[r]
    INF = jnp.float32(jnp.inf)
    sv0 = jnp.full((R, KNB), INF, jnp.float32)
    si0 = lax.broadcasted_iota(jnp.int32, (R, KNB), 1).astype(jnp.float32)
    W2 = (_C + KNB + KNB) // 2

    def tile(t, S):
        sv, si = S
        j0 = pl.multiple_of(lo + t * _C, 128)
        xc = xf_ref[pl.ds(j0, _C), :]
        n2c = n2t_ref[:, pl.ds(j0, _C)]
        mm = lax.dot_general(xr, xc, (((1,), (1,)), ((), ())),
                             preferred_element_type=jnp.float32)
        d = (n2r + n2c) - 2.0 * mm
        ci = lax.broadcasted_iota(jnp.int32, (R, _C), 1) + j0
        ok = (ci >= rlo) & (ci < rhi)
        d = jnp.where(ok, d, INF)
        cif = ci.astype(jnp.float32)
        V = jnp.concatenate([d, sv, jnp.full((R, KNB), INF, jnp.float32)],
                            axis=1)
        I = jnp.concatenate([cif, si, jnp.full((R, KNB), 3.0e7, jnp.float32)],
                            axis=1)
        A, B = V[:, :W2], V[:, W2:]
        IA, IB = I[:, :W2], I[:, W2:]
        P = A <= B
        M = jnp.where(P, A, B)
        IM = jnp.where(P, IA, IB)
        L = jnp.where(P, B, A)
        IL = jnp.where(P, IB, IA)
        vs, ixs = [], []
        for _ in range(KNB):
            v = jnp.min(M, axis=1, keepdims=True)
            eq = M == v
            ix = jnp.min(jnp.where(eq, IM, 3.0e7), axis=1, keepdims=True)
            kill = IM == ix
            M = jnp.where(kill, L, M)
            IM = jnp.where(kill, IL, IM)
            L = jnp.where(kill, INF, L)
            vs.append(v)
            ixs.append(ix)
        return jnp.concatenate(vs, axis=1), jnp.concatenate(ixs, axis=1)

    _, si = lax.fori_loop(0, nt, tile, (sv0, si0))
    idx_ref[...] = si.astype(jnp.int32)


def _knn_call(x, xrows, rlo, rhi, lo128, nt):
    NP, D = x.shape
    NR = xrows.shape[0]
    G = NR // _R
    n2 = jnp.sum(x * x, axis=1)
    n2r = jnp.sum(xrows * xrows, axis=1)[:, None]
    n2t = n2[None, :]
    grid_spec = pltpu.PrefetchScalarGridSpec(
        num_scalar_prefetch=2,
        grid=(G,),
        in_specs=[
            pl.BlockSpec((_R, D), lambda r, lo_, nt_: (r, 0)),
            pl.BlockSpec((_R, 1), lambda r, lo_, nt_: (r, 0)),
            pl.BlockSpec((_R, 1), lambda r, lo_, nt_: (r, 0)),
            pl.BlockSpec((_R, 1), lambda r, lo_, nt_: (r, 0)),
            pl.BlockSpec((NP, D), lambda r, lo_, nt_: (0, 0)),
            pl.BlockSpec((1, NP), lambda r, lo_, nt_: (0, 0)),
        ],
        out_specs=[
            pl.BlockSpec((_R, KNB), lambda r, lo_, nt_: (r, 0)),
        ],
    )
    return pl.pallas_call(
        _knn_body,
        grid_spec=grid_spec,
        out_shape=[
            jax.ShapeDtypeStruct((NR, KNB), jnp.int32),
        ],
    )(lo128, nt, xrows, rlo, rhi, n2r, x, n2t)[0]


def _sc_gather(tab, idx_flat):
    (E,) = idx_flat.shape
    _, D = tab.shape
    NW = 32
    epw = E // NW
    CH = 128
    nch = epw // CH
    mesh = plsc.VectorSubcoreMesh(core_axis_name="c", subcore_axis_name="s")

    NBUF = 6 if D <= 64 else 5
    while nch % NBUF:
        NBUF -= 1
    ngrp = nch // NBUF

    @functools.partial(
        pl.kernel,
        out_type=jax.ShapeDtypeStruct((E, D), jnp.float32),
        mesh=mesh,
        scratch_types=[
            pltpu.VMEM((epw,), jnp.int32),
            pltpu.VMEM((NBUF, CH, D), jnp.float32),
            pltpu.SemaphoreType.DMA,
        ],
    )
    def gk(idx_hbm, tab_hbm, out_hbm, idx_v, rows_v, sem):
        wid = lax.axis_index("s") * 2 + lax.axis_index("c")
        base = wid * epw
        pltpu.sync_copy(idx_hbm.at[pl.ds(base, epw)], idx_v)

        def body(g, carry):
            ch0 = g * NBUF
            copies = [
                pltpu.async_copy(
                    tab_hbm.at[idx_v.at[pl.ds((ch0 + b) * CH, CH)]],
                    rows_v.at[b], sem)
                for b in range(NBUF)
            ]
            for b in range(NBUF):
                copies[b].wait()
            for b in range(NBUF):
                pltpu.sync_copy(rows_v.at[b],
                                out_hbm.at[pl.ds(base + (ch0 + b) * CH, CH)])
            return carry

        lax.fori_loop(0, ngrp, body, 0)

    return gk(idx_flat, tab)


def _edge1_body(e_ref, xi_ref, w1_ref, b1_ref, w2_ref, b2_ref, w3a_ref,
                b3_ref, w3u_ref, x1_ref, c2_ref, y2_ref):
    xi = xi_ref[...]
    RP = xi.shape[0]
    feats = []
    for k in range(KNB):
        xj = e_ref[k][:, :3]
        feats.append(jnp.concatenate([xi, xj - xi], axis=1))
    F = jnp.concatenate(feats, axis=0)
    H1 = jnp.maximum(
        jnp.dot(F, w1_ref[...], preferred_element_type=jnp.float32)
        + b1_ref[...], 0.0)
    H = jnp.dot(H1, w2_ref[...], preferred_element_type=jnp.float32)
    m = H[0:RP]
    for k in range(1, KNB):
        m = jnp.maximum(m, H[k * RP:(k + 1) * RP])
    x1 = m + b2_ref[...]
    x1_ref[...] = x1
    c2_ref[...] = (jnp.dot(x1, w3a_ref[...], preferred_element_type=jnp.float32)
                   + b3_ref[...])
    y2_ref[...] = jnp.dot(x1, w3u_ref[...], preferred_element_type=jnp.float32)


def _final_body(g2_ref, c_ref, x1_ref, br_ref, wla_ref, wlb_ref, bl_ref, out_ref):
    p = pl.program_id(0)
    m = g2_ref[0]
    for k in range(1, KNB):
        m = jnp.maximum(m, g2_ref[k])
    x2 = c_ref[...] + m
    h = (jnp.dot(x1_ref[...], wla_ref[...], preferred_element_type=jnp.float32)
         + jnp.dot(x2, wlb_ref[...], preferred_element_type=jnp.float32)
         + bl_ref[...])
    NINF = jnp.float32(-jnp.inf)

    @pl.when(p == 0)
    def _():
        out_ref[...] = jnp.full(out_ref.shape, NINF, jnp.float32)

    b = br_ref[...]
    rows = [jnp.max(jnp.where(b == g, h, NINF), axis=0, keepdims=True)
            for g in range(NGR)]
    out_ref[...] = jnp.maximum(out_ref[...], jnp.concatenate(rows, axis=0))


def _edge1_call(e1_h, pos_h, W1, b1, W2, b2, W3a, b3, W3u):
    NH = pos_h.shape[0]
    return pl.pallas_call(
        _edge1_body,
        grid=(NH // _RP,),
        in_specs=[
            pl.BlockSpec((KNB, _RP, 128), lambda p: (0, p, 0)),
            pl.BlockSpec((_RP, 3), lambda p: (p, 0)),
            pl.BlockSpec((6, 64), lambda p: (0, 0)),
            pl.BlockSpec((1, 64), lambda p: (0, 0)),
            pl.BlockSpec((64, 64), lambda p: (0, 0)),
            pl.BlockSpec((1, 64), lambda p: (0, 0)),
            pl.BlockSpec((64, 128), lambda p: (0, 0)),
            pl.BlockSpec((1, 128), lambda p: (0, 0)),
            pl.BlockSpec((64, 128), lambda p: (0, 0)),
        ],
        out_specs=[
            pl.BlockSpec((_RP, 64), lambda p: (p, 0)),
            pl.BlockSpec((_RP, 128), lambda p: (p, 0)),
            pl.BlockSpec((_RP, 128), lambda p: (p, 0)),
        ],
        out_shape=[
            jax.ShapeDtypeStruct((NH, 64), jnp.float32),
            jax.ShapeDtypeStruct((NH, 128), jnp.float32),
            jax.ShapeDtypeStruct((NH, 128), jnp.float32),
        ],
    )(e1_h.reshape(KNB, NH, 128), pos_h, W1, b1[None, :], W2, b2[None, :],
      W3a, b3[None, :], W3u)


def _final_call(g2_h, c2_h, x1_h, batch_h, Wl, bl):
    NH = x1_h.shape[0]
    return pl.pallas_call(
        _final_body,
        grid=(NH // _RP,),
        in_specs=[
            pl.BlockSpec((KNB, _RP, 128), lambda p: (0, p, 0)),
            pl.BlockSpec((_RP, 128), lambda p: (p, 0)),
            pl.BlockSpec((_RP, 64), lambda p: (p, 0)),
            pl.BlockSpec((_RP, 1), lambda p: (p, 0)),
            pl.BlockSpec((64, 128), lambda p: (0, 0)),
            pl.BlockSpec((128, 128), lambda p: (0, 0)),
            pl.BlockSpec((1, 128), lambda p: (0, 0)),
        ],
        out_specs=pl.BlockSpec((NGR, 128), lambda p: (0, 0)),
        out_shape=jax.ShapeDtypeStruct((NGR, 128), jnp.float32),
    )(g2_h.reshape(KNB, NH, 128), c2_h, x1_h, batch_h[:, None], Wl[:64],
      Wl[64:], bl[None, :])


def kernel(pos, batch, W1, b1, W2, b2, W3, b3, Wl, bl, train=0):
    N = pos.shape[0]
    H = N // 2
    batch = batch.astype(jnp.int32)
    starts = jnp.searchsorted(
        batch, jnp.arange(NGR + 1, dtype=jnp.int32), side="left").astype(jnp.int32)
    rlo = starts[batch][:, None]
    rhi = starts[batch + 1][:, None]
    lo_blk = starts[batch[::_R]]
    hi_blk = starts[batch[_R - 1::_R] + 1]
    lo128 = (lo_blk // 128) * 128
    nt = (hi_blk - lo128 + _C - 1) // _C
    GH = (N // _R) // 2

    def hsl(a, h):
        return a[h * H:(h + 1) * H]


    W1a = W1[:3] - W1[3:]
    W3a = W3[:64] - W3[64:]
    W3u = W3[64:]
    xpad1 = jnp.concatenate([pos, jnp.zeros((_C, 3), pos.dtype)], axis=0)
    pos_pad = jnp.concatenate([pos, jnp.zeros((N, 125), pos.dtype)], axis=1)

    idx1 = [_knn_call(xpad1, hsl(pos, h), hsl(rlo, h), hsl(rhi, h),
                      lo128[h * GH:(h + 1) * GH], nt[h * GH:(h + 1) * GH])
            for h in (0, 1)]
    e1 = [_sc_gather(pos_pad, idx1[h].T.reshape(-1)) for h in (0, 1)]
    ed = [_edge1_call(e1[h], hsl(pos, h), W1, b1, W2, b2, W3a, b3, W3u)
          for h in (0, 1)]
    x1 = jnp.concatenate([ed[0][0], ed[1][0]], axis=0)
    y2 = jnp.concatenate([ed[0][2], ed[1][2]], axis=0)

    xpad2 = jnp.concatenate([x1, jnp.zeros((_C, 64), x1.dtype)], axis=0)
    idx2 = [_knn_call(xpad2, hsl(x1, h), hsl(rlo, h), hsl(rhi, h),
                      lo128[h * GH:(h + 1) * GH], nt[h * GH:(h + 1) * GH])
            for h in (0, 1)]
    g2 = [_sc_gather(y2, idx2[h].T.reshape(-1)) for h in (0, 1)]

    outs = [_final_call(g2[h], ed[h][1], ed[h][0], hsl(batch, h), Wl, bl)
            for h in (0, 1)]
    return jnp.maximum(outs[0], outs[1])


# speedup vs baseline: 1.3625x; 1.1089x over previous
import functools

import jax
import jax.numpy as jnp
from jax import lax
from jax.experimental import pallas as pl
from jax.experimental.pallas import tpu as pltpu
from jax.experimental.pallas import tpu_sc as plsc

KNB = 20
NGR = 8
_R = 256
_C = 640
_RP = 128
_BIG = 2**30


def _knn_body(lo_ref, nt_ref, xr_ref, rlo_ref, rhi_ref, n2r_ref, xf_ref,
              n2t_ref, idx_ref):
    r = pl.program_id(0)
    xr = xr_ref[...]
    R = xr.shape[0]
    n2r = n2r_ref[...]
    rlo = rlo_ref[...]
    rhi = rhi_ref[...]
    lo = lo_ref[r]
    nt = nt_ref[r]
    INF = jnp.float32(jnp.inf)
    sv0 = jnp.full((R, KNB), INF, jnp.float32)
    si0 = lax.broadcasted_iota(jnp.int32, (R, KNB), 1).astype(jnp.float32)
    W2 = (_C + KNB + KNB) // 2

    def tile(t, S):
        sv, si = S
        j0 = pl.multiple_of(lo + t * _C, 128)
        xc = xf_ref[pl.ds(j0, _C), :]
        n2c = n2t_ref[:, pl.ds(j0, _C)]
        mm = lax.dot_general(xr, xc, (((1,), (1,)), ((), ())),
                             preferred_element_type=jnp.float32)
        d = (n2r + n2c) - 2.0 * mm
        ci = lax.broadcasted_iota(jnp.int32, (R, _C), 1) + j0
        ok = (ci >= rlo) & (ci < rhi)
        d = jnp.where(ok, d, INF)
        cif = ci.astype(jnp.float32)
        V = jnp.concatenate([d, sv, jnp.full((R, KNB), INF, jnp.float32)],
                            axis=1)
        I = jnp.concatenate([cif, si, jnp.full((R, KNB), 3.0e7, jnp.float32)],
                            axis=1)
        A, B = V[:, :W2], V[:, W2:]
        IA, IB = I[:, :W2], I[:, W2:]
        P = A <= B
        M = jnp.where(P, A, B)
        IM = jnp.where(P, IA, IB)
        L = jnp.where(P, B, A)
        IL = jnp.where(P, IB, IA)
        vs, ixs = [], []
        for _ in range(KNB):
            v = jnp.min(M, axis=1, keepdims=True)
            eq = M == v
            ix = jnp.min(jnp.where(eq, IM, 3.0e7), axis=1, keepdims=True)
            kill = IM == ix
            M = jnp.where(kill, L, M)
            IM = jnp.where(kill, IL, IM)
            L = jnp.where(kill, INF, L)
            vs.append(v)
            ixs.append(ix)
        return jnp.concatenate(vs, axis=1), jnp.concatenate(ixs, axis=1)

    _, si = lax.fori_loop(0, nt, tile, (sv0, si0))
    idx_ref[...] = si.astype(jnp.int32)


def _knn_call(x, xrows, rlo, rhi, lo128, nt):
    NP, D = x.shape
    NR = xrows.shape[0]
    G = NR // _R
    n2 = jnp.sum(x * x, axis=1)
    n2r = jnp.sum(xrows * xrows, axis=1)[:, None]
    n2t = n2[None, :]
    grid_spec = pltpu.PrefetchScalarGridSpec(
        num_scalar_prefetch=2,
        grid=(G,),
        in_specs=[
            pl.BlockSpec((_R, D), lambda r, lo_, nt_: (r, 0)),
            pl.BlockSpec((_R, 1), lambda r, lo_, nt_: (r, 0)),
            pl.BlockSpec((_R, 1), lambda r, lo_, nt_: (r, 0)),
            pl.BlockSpec((_R, 1), lambda r, lo_, nt_: (r, 0)),
            pl.BlockSpec((NP, D), lambda r, lo_, nt_: (0, 0)),
            pl.BlockSpec((1, NP), lambda r, lo_, nt_: (0, 0)),
        ],
        out_specs=[
            pl.BlockSpec((_R, KNB), lambda r, lo_, nt_: (r, 0)),
        ],
    )
    return pl.pallas_call(
        _knn_body,
        grid_spec=grid_spec,
        out_shape=[
            jax.ShapeDtypeStruct((NR, KNB), jnp.int32),
        ],
    )(lo128, nt, xrows, rlo, rhi, n2r, x, n2t)[0]


def _sc_gather(tab, idx_flat):
    (E,) = idx_flat.shape
    _, D = tab.shape
    NW = 32
    epw = E // NW
    CH = 128
    nch = epw // CH
    mesh = plsc.VectorSubcoreMesh(core_axis_name="c", subcore_axis_name="s")

    NBUF = 6 if D <= 64 else 5
    while nch % NBUF:
        NBUF -= 1
    ngrp = nch // NBUF

    @functools.partial(
        pl.kernel,
        out_type=jax.ShapeDtypeStruct((E, D), jnp.float32),
        mesh=mesh,
        scratch_types=[
            pltpu.VMEM((epw,), jnp.int32),
            pltpu.VMEM((NBUF, CH, D), jnp.float32),
            pltpu.SemaphoreType.DMA,
        ],
    )
    def gk(idx_hbm, tab_hbm, out_hbm, idx_v, rows_v, sem):
        wid = lax.axis_index("s") * 2 + lax.axis_index("c")
        base = wid * epw
        pltpu.sync_copy(idx_hbm.at[pl.ds(base, epw)], idx_v)

        def body(g, carry):
            ch0 = g * NBUF
            copies = [
                pltpu.async_copy(
                    tab_hbm.at[idx_v.at[pl.ds((ch0 + b) * CH, CH)]],
                    rows_v.at[b], sem)
                for b in range(NBUF)
            ]
            for b in range(NBUF):
                copies[b].wait()
            for b in range(NBUF):
                pltpu.sync_copy(rows_v.at[b],
                                out_hbm.at[pl.ds(base + (ch0 + b) * CH, CH)])
            return carry

        lax.fori_loop(0, ngrp, body, 0)

    return gk(idx_flat, tab)


def _edge1_body(e_ref, xi_ref, w1_ref, b1_ref, w2_ref, b2_ref, w3a_ref,
                b3_ref, w3u_ref, x1_ref, c2_ref, y2_ref):
    xi = xi_ref[...]
    RP = xi.shape[0]
    feats = []
    for k in range(KNB):
        xj = e_ref[k][:, :3]
        feats.append(jnp.concatenate([xi, xj - xi], axis=1))
    F = jnp.concatenate(feats, axis=0)
    H1 = jnp.maximum(
        jnp.dot(F, w1_ref[...], preferred_element_type=jnp.float32)
        + b1_ref[...], 0.0)
    H = jnp.dot(H1, w2_ref[...], preferred_element_type=jnp.float32)
    m = H[0:RP]
    for k in range(1, KNB):
        m = jnp.maximum(m, H[k * RP:(k + 1) * RP])
    x1 = m + b2_ref[...]
    x1_ref[...] = x1
    c2_ref[...] = (jnp.dot(x1, w3a_ref[...], preferred_element_type=jnp.float32)
                   + b3_ref[...])
    y2_ref[...] = jnp.dot(x1, w3u_ref[...], preferred_element_type=jnp.float32)


def _final_body(g2_ref, c_ref, x1_ref, br_ref, wla_ref, wlb_ref, bl_ref, out_ref):
    p = pl.program_id(0)
    m = g2_ref[0]
    for k in range(1, KNB):
        m = jnp.maximum(m, g2_ref[k])
    x2 = c_ref[...] + m
    h = (jnp.dot(x1_ref[...], wla_ref[...], preferred_element_type=jnp.float32)
         + jnp.dot(x2, wlb_ref[...], preferred_element_type=jnp.float32)
         + bl_ref[...])
    NINF = jnp.float32(-jnp.inf)

    @pl.when(p == 0)
    def _():
        out_ref[...] = jnp.full(out_ref.shape, NINF, jnp.float32)

    b = br_ref[...]
    rows = [jnp.max(jnp.where(b == g, h, NINF), axis=0, keepdims=True)
            for g in range(NGR)]
    out_ref[...] = jnp.maximum(out_ref[...], jnp.concatenate(rows, axis=0))


def _edge1_call(e1_h, pos_h, W1, b1, W2, b2, W3a, b3, W3u):
    NH = pos_h.shape[0]
    return pl.pallas_call(
        _edge1_body,
        grid=(NH // _RP,),
        in_specs=[
            pl.BlockSpec((KNB, _RP, 128), lambda p: (0, p, 0)),
            pl.BlockSpec((_RP, 3), lambda p: (p, 0)),
            pl.BlockSpec((6, 64), lambda p: (0, 0)),
            pl.BlockSpec((1, 64), lambda p: (0, 0)),
            pl.BlockSpec((64, 64), lambda p: (0, 0)),
            pl.BlockSpec((1, 64), lambda p: (0, 0)),
            pl.BlockSpec((64, 128), lambda p: (0, 0)),
            pl.BlockSpec((1, 128), lambda p: (0, 0)),
            pl.BlockSpec((64, 128), lambda p: (0, 0)),
        ],
        out_specs=[
            pl.BlockSpec((_RP, 64), lambda p: (p, 0)),
            pl.BlockSpec((_RP, 128), lambda p: (p, 0)),
            pl.BlockSpec((_RP, 128), lambda p: (p, 0)),
        ],
        out_shape=[
            jax.ShapeDtypeStruct((NH, 64), jnp.float32),
            jax.ShapeDtypeStruct((NH, 128), jnp.float32),
            jax.ShapeDtypeStruct((NH, 128), jnp.float32),
        ],
    )(e1_h.reshape(KNB, NH, 128), pos_h, W1, b1[None, :], W2, b2[None, :],
      W3a, b3[None, :], W3u)


def _final_call(g2_h, c2_h, x1_h, batch_h, Wl, bl):
    NH = x1_h.shape[0]
    return pl.pallas_call(
        _final_body,
        grid=(NH // _RP,),
        in_specs=[
            pl.BlockSpec((KNB, _RP, 128), lambda p: (0, p, 0)),
            pl.BlockSpec((_RP, 128), lambda p: (p, 0)),
            pl.BlockSpec((_RP, 64), lambda p: (p, 0)),
            pl.BlockSpec((_RP, 1), lambda p: (p, 0)),
            pl.BlockSpec((64, 128), lambda p: (0, 0)),
            pl.BlockSpec((128, 128), lambda p: (0, 0)),
            pl.BlockSpec((1, 128), lambda p: (0, 0)),
        ],
        out_specs=pl.BlockSpec((NGR, 128), lambda p: (0, 0)),
        out_shape=jax.ShapeDtypeStruct((NGR, 128), jnp.float32),
    )(g2_h.reshape(KNB, NH, 128), c2_h, x1_h, batch_h[:, None], Wl[:64],
      Wl[64:], bl[None, :])


def kernel(pos, batch, W1, b1, W2, b2, W3, b3, Wl, bl, train=0):
    N = pos.shape[0]
    H = N // 2
    batch = batch.astype(jnp.int32)
    starts = jnp.searchsorted(
        batch, jnp.arange(NGR + 1, dtype=jnp.int32), side="left").astype(jnp.int32)
    rlo = starts[batch][:, None]
    rhi = starts[batch + 1][:, None]
    lo_blk = starts[batch[::_R]]
    hi_blk = starts[batch[_R - 1::_R] + 1]
    lo128 = (lo_blk // 128) * 128
    nt = (hi_blk - lo128 + _C - 1) // _C
    GH = (N // _R) // 2

    def hsl(a, h):
        return a[h * H:(h + 1) * H]


    W1a = W1[:3] - W1[3:]
    W3a = W3[:64] - W3[64:]
    W3u = W3[64:]
    xpad1 = jnp.concatenate([pos, jnp.zeros((_C, 3), pos.dtype)], axis=0)
    pos_pad = jnp.concatenate([pos, jnp.zeros((N, 125), pos.dtype)], axis=1)

    idx1 = [_knn_call(xpad1, hsl(pos, h), hsl(rlo, h), hsl(rhi, h),
                      lo128[h * GH:(h + 1) * GH], nt[h * GH:(h + 1) * GH])
            for h in (0, 1)]
    e1 = [_sc_gather(pos_pad, idx1[h].T.reshape(-1)) for h in (0, 1)]
    ed = [_edge1_call(e1[h], hsl(pos, h), W1, b1, W2, b2, W3a, b3, W3u)
          for h in (0, 1)]
    x1 = jnp.concatenate([ed[0][0], ed[1][0]], axis=0)
    y2 = jnp.concatenate([ed[0][2], ed[1][2]], axis=0)

    xpad2 = jnp.concatenate([x1, jnp.zeros((_C, 64), x1.dtype)], axis=0)
    idx2 = [_knn_call(xpad2, hsl(x1, h), hsl(rlo, h), hsl(rhi, h),
                      lo128[h * GH:(h + 1) * GH], nt[h * GH:(h + 1) * GH])
            for h in (0, 1)]
    g2 = [_sc_gather(y2, idx2[h].T.reshape(-1)) for h in (0, 1)]

    outs = [_final_call(g2[h], ed[h][1], ed[h][0], hsl(batch, h), Wl, bl)
            for h in (0, 1)]
    return jnp.maximum(outs[0], outs[1])
